# Initial kernel scaffold; baseline (speedup 1.0000x reference)
import jax
import jax.numpy as jnp
from jax import lax
from jax.experimental import pallas as pl

V = 1000
VP = 1024
B = 16384
BK = 1024
NBLK = B // BK


def _body(ctx_ref, tgt_ref, tri_ref, uni_ref, big_ref,
          oh_out, uni_out, bi_out, acc_ref):
    i = pl.program_id(0)

    tri = tri_ref[...]
    lane = lax.broadcasted_iota(jnp.int32, (BK, V), 1)
    oh_out[...] = (lane == tri).astype(jnp.float32)

    @pl.when(i == 0)
    def _init():
        acc_ref[...] = jnp.zeros((VP, VP), jnp.float32)

    ctx = ctx_ref[0]
    tgt = tgt_ref[...]
    sub = lax.broadcasted_iota(jnp.int32, (VP, BK), 0)
    lane2 = lax.broadcasted_iota(jnp.int32, (BK, VP), 1)
    ohT_ctx = (sub == ctx).astype(jnp.bfloat16)
    oh_tgt = (lane2 == tgt).astype(jnp.bfloat16)
    acc_ref[...] += jnp.dot(ohT_ctx, oh_tgt,
                            preferred_element_type=jnp.float32)

    @pl.when(i == NBLK - 1)
    def _fin():
        acc = acc_ref[...]
        bi_out[...] = acc[:V, :V] + big_ref[...]
        uni = jnp.sum(acc, axis=1, keepdims=True)
        uni_out[...] = uni[:V] + uni_ref[...]


def kernel(batch, unigrams, bigrams, w):
    batch = batch.astype(jnp.int32)
    ctx_rows = batch.reshape(NBLK, 1, BK)
    pad_tgt = jnp.full((1,), VP - 1, jnp.int32)
    tgt_col = jnp.concatenate([batch[1:], pad_tgt]).reshape(B, 1)
    tri_col = jnp.concatenate([batch[2:], jnp.zeros((2,), jnp.int32)]).reshape(B, 1)

    oh_tri, uni_new, bi_new = pl.pallas_call(
        _body,
        grid=(NBLK,),
        in_specs=[
            pl.BlockSpec((1, 1, BK), lambda i: (i, 0, 0)),
            pl.BlockSpec((BK, 1), lambda i: (i, 0)),
            pl.BlockSpec((BK, 1), lambda i: (i, 0)),
            pl.BlockSpec((V, 1), lambda i: (0, 0)),
            pl.BlockSpec((V, V), lambda i: (0, 0)),
        ],
        out_specs=[
            pl.BlockSpec((BK, V), lambda i: (i, 0)),
            pl.BlockSpec((V, 1), lambda i: (0, 0)),
            pl.BlockSpec((V, V), lambda i: (0, 0)),
        ],
        out_shape=[
            jax.ShapeDtypeStruct((B - 2, V), jnp.float32),
            jax.ShapeDtypeStruct((V, 1), jnp.float32),
            jax.ShapeDtypeStruct((V, V), jnp.float32),
        ],
        scratch_shapes=[pltpu.VMEM((VP, VP), jnp.float32)],
    )(ctx_rows, tgt_col, tri_col, unigrams, bigrams)

    return (uni_new, bi_new, oh_tri)


from jax.experimental.pallas import tpu as pltpu


# speedup vs baseline: 9.1874x; 9.1874x over previous
import jax
import jax.numpy as jnp
from jax import lax
from jax.experimental import pallas as pl
from jax.experimental.pallas import tpu as pltpu

V = 1000
VP = 1024
B = 16384
BK = 1024
NBLK = B // BK


def _body(ctx_ref, tgt_ref, tri_ref, uni_ref, big_ref,
          oh_out, uni_out, bi_out, acc_ref):
    i = pl.program_id(0)

    tri = tri_ref[...]
    lane = lax.broadcasted_iota(jnp.int32, (BK, V), 1)
    oh_out[...] = (lane == tri).astype(jnp.float32)

    @pl.when(i == 0)
    def _init():
        acc_ref[...] = jnp.zeros((VP, VP), jnp.float32)

    ctx = ctx_ref[0]
    tgt = tgt_ref[...]
    sub = lax.broadcasted_iota(jnp.int32, (VP, BK), 0)
    lane2 = lax.broadcasted_iota(jnp.int32, (BK, VP), 1)
    ohT_ctx = (sub == ctx).astype(jnp.bfloat16)
    oh_tgt = (lane2 == tgt).astype(jnp.bfloat16)
    acc_ref[...] += jnp.dot(ohT_ctx, oh_tgt,
                            preferred_element_type=jnp.float32)

    @pl.when(i == NBLK - 1)
    def _fin():
        acc = acc_ref[...]
        bi_out[...] = acc[:V, :V] + big_ref[...]
        uni = jnp.sum(acc, axis=1, keepdims=True)
        uni_out[...] = uni[:V] + uni_ref[...]


def kernel(batch, unigrams, bigrams, w):
    batch = batch.astype(jnp.int32)
    ctx_rows = batch.reshape(NBLK, 1, BK)
    pad_tgt = jnp.full((1,), VP - 1, jnp.int32)
    tgt_col = jnp.concatenate([batch[1:], pad_tgt]).reshape(B, 1)
    tri_col = jnp.concatenate([batch[2:], jnp.zeros((2,), jnp.int32)]).reshape(B, 1)

    oh_tri, uni_new, bi_new = pl.pallas_call(
        _body,
        grid=(NBLK,),
        in_specs=[
            pl.BlockSpec((1, 1, BK), lambda i: (i, 0, 0)),
            pl.BlockSpec((BK, 1), lambda i: (i, 0)),
            pl.BlockSpec((BK, 1), lambda i: (i, 0)),
            pl.BlockSpec((V, 1), lambda i: (0, 0)),
            pl.BlockSpec((V, V), lambda i: (0, 0)),
        ],
        out_specs=[
            pl.BlockSpec((BK, V), lambda i: (i, 0)),
            pl.BlockSpec((V, 1), lambda i: (0, 0)),
            pl.BlockSpec((V, V), lambda i: (0, 0)),
        ],
        out_shape=[
            jax.ShapeDtypeStruct((B - 2, V), jnp.float32),
            jax.ShapeDtypeStruct((V, 1), jnp.float32),
            jax.ShapeDtypeStruct((V, V), jnp.float32),
        ],
        scratch_shapes=[pltpu.VMEM((VP, VP), jnp.float32)],
    )(ctx_rows, tgt_col, tri_col, unigrams, bigrams)

    return (uni_new, bi_new, oh_tri)
